# Initial kernel scaffold; baseline (speedup 1.0000x reference)
import jax
import jax.numpy as jnp
from jax.experimental import pallas as pl


def kernel(mu, private_label):
    raise NotImplementedError("write your pallas kernel here")


# speedup vs baseline: 9.5737x; 9.5737x over previous
import jax
import jax.numpy as jnp
from jax import lax
from jax.experimental import pallas as pl

SIGMA_ = 1.0
C_ = 10
CP_ = 16


def _stats_body(lab_ref, mu_ref, sums_ref, sqs_ref, cnt_ref):
    i = pl.program_id(0)
    B = mu_ref.shape[0]
    labs = lab_ref[0]
    labs_b = jnp.broadcast_to(labs, (CP_, B))
    cls = lax.broadcasted_iota(jnp.int32, (CP_, B), 0)
    oh = (labs_b == cls).astype(jnp.float32)
    m = mu_ref[...]
    dn = (((1,), (0,)), ((), ()))
    s = lax.dot_general(oh, m, dn, preferred_element_type=jnp.float32, precision=lax.Precision.HIGHEST)
    q = lax.dot_general(oh, m * m, dn, preferred_element_type=jnp.float32, precision=lax.Precision.HIGHEST)
    ones = jnp.ones((B, 128), jnp.float32)
    c = lax.dot_general(oh, ones, dn, preferred_element_type=jnp.float32, precision=lax.Precision.HIGHEST)

    @pl.when(i == 0)
    def _():
        sums_ref[...] = s
        sqs_ref[...] = q
        cnt_ref[...] = c

    @pl.when(i != 0)
    def _():
        sums_ref[...] += s
        sqs_ref[...] += q
        cnt_ref[...] += c


def _finalize_body(sums_ref, sqs_ref, cnt_ref, out_ref):
    counts = cnt_ref[...]
    sums = sums_ref[...]
    sqs = sqs_ref[...]
    safe = jnp.maximum(counts, 1.0)
    muF = sums / safe
    SigF = sqs / safe - muF * muF + SIGMA_
    SigF = jnp.maximum(SigF, 1e-6)

    logS = jnp.log(SigF)
    R2 = 1.0 / SigF
    onesr = jnp.ones((1, 128), jnp.float32)
    dnT = (((1,), (1,)), ((), ()))
    f32 = jnp.float32
    ld_i = lax.dot_general(logS, onesr, dnT, preferred_element_type=f32, precision=lax.Precision.HIGHEST)
    ld_j = lax.dot_general(onesr, logS, dnT, preferred_element_type=f32, precision=lax.Precision.HIGHEST)
    A = lax.dot_general(SigF, R2, dnT, preferred_element_type=f32, precision=lax.Precision.HIGHEST)
    m2 = muF * muF
    B1 = lax.dot_general(m2, R2, dnT, preferred_element_type=f32, precision=lax.Precision.HIGHEST)
    B2 = lax.dot_general(muF, muF * R2, dnT, preferred_element_type=f32, precision=lax.Precision.HIGHEST)
    t_j = lax.dot_general(onesr, m2 * R2, dnT, preferred_element_type=f32, precision=lax.Precision.HIGHEST)

    D = 128.0
    kl = 0.5 * (ld_j - ld_i + A + B1 - 2.0 * B2 + t_j - D)

    pres = (counts > 0.0).astype(f32)
    pres_i = lax.dot_general(pres, onesr / D, dnT, preferred_element_type=f32, precision=lax.Precision.HIGHEST)
    pres_j = lax.dot_general(onesr / D, pres, dnT, preferred_element_type=f32, precision=lax.Precision.HIGHEST)
    ri = lax.broadcasted_iota(jnp.int32, (CP_, CP_), 0)
    ci = lax.broadcasted_iota(jnp.int32, (CP_, CP_), 1)
    off_diag = (ri != ci).astype(f32)
    mask = pres_i * pres_j * off_diag
    key_num = jnp.sum(pres_i)
    denom = jnp.maximum(key_num * (key_num - 1.0), 1.0)
    loss = jnp.sum(kl * mask) / denom
    out_ref[...] = jnp.broadcast_to(loss, (1, 1))


def _gmm_loss(mu, labels_3d):
    G = labels_3d.shape[0]
    B = labels_3d.shape[2]
    sums, sqs, cnt = pl.pallas_call(
        _stats_body,
        grid=(G,),
        in_specs=[
            pl.BlockSpec((1, 1, B), lambda i: (i, 0, 0)),
            pl.BlockSpec((B, 128), lambda i: (i, 0)),
        ],
        out_specs=[
            pl.BlockSpec((CP_, 128), lambda i: (0, 0)),
            pl.BlockSpec((CP_, 128), lambda i: (0, 0)),
            pl.BlockSpec((CP_, 128), lambda i: (0, 0)),
        ],
        out_shape=[
            jax.ShapeDtypeStruct((CP_, 128), jnp.float32),
            jax.ShapeDtypeStruct((CP_, 128), jnp.float32),
            jax.ShapeDtypeStruct((CP_, 128), jnp.float32),
        ],
    )(labels_3d, mu)

    loss = pl.pallas_call(
        _finalize_body,
        out_shape=jax.ShapeDtypeStruct((1, 1), jnp.float32),
    )(sums, sqs, cnt)
    return loss[0, 0]


def kernel(mu, private_label):
    N, D = mu.shape
    B = 2048
    G = N // B
    labels = private_label.astype(jnp.int32).reshape(G, 1, B)
    return _gmm_loss(mu, labels)
